# initial kernel scaffold (unmeasured)
import jax
import jax.numpy as jnp
from jax import lax
from jax.experimental import pallas as pl
from jax.experimental.pallas import tpu as pltpu

N_DEV = 8
M_PER = 512
K = 4096
N_PER = 1024
TILE_N = 256
N_T = N_PER // TILE_N


def kernel(x, w_mat):
    def body(x_ref, w_hbm, out_ref, x_bf, w_buf, send_buf, recv_buf,
             w_sems, send_sems, recv_sems):
        me = lax.axis_index("i")

        barrier = pltpu.get_barrier_semaphore()
        for s in range(1, N_DEV):
            pl.semaphore_signal(
                barrier, inc=1,
                device_id=((me + s) % N_DEV,),
                device_id_type=pl.DeviceIdType.MESH,
            )
        pl.semaphore_wait(barrier, N_DEV - 1)

        x_bf[...] = x_ref[...].astype(jnp.bfloat16)

        seq = list(range(1, N_DEV)) + [0]
        tiles = [(s, t) for s in seq for t in range(N_T)]

        def w_col(s, t):
            return ((me + s) % N_DEV) * N_PER + t * TILE_N

        def w_load(k, slot):
            s, t = tiles[k]
            return pltpu.make_async_copy(
                w_hbm.at[:, pl.ds(w_col(s, t), TILE_N)],
                w_buf.at[slot],
                w_sems.at[slot],
            )

        def send_rdma(s, t):
            return pltpu.make_async_remote_copy(
                src_ref=send_buf.at[s, :, pl.ds(t * TILE_N, TILE_N)],
                dst_ref=recv_buf.at[s, :, pl.ds(t * TILE_N, TILE_N)],
                send_sem=send_sems.at[s, t],
                recv_sem=recv_sems.at[s, t],
                device_id=((me + s) % N_DEV,),
                device_id_type=pl.DeviceIdType.MESH,
            )

        w_load(0, 0).start()
        for k, (s, t) in enumerate(tiles):
            slot = k % 2
            if k + 1 < len(tiles):
                w_load(k + 1, (k + 1) % 2).start()
            w_load(k, slot).wait()
            res = jnp.dot(
                x_bf[...],
                w_buf[slot].astype(jnp.bfloat16),
                preferred_element_type=jnp.float32,
            )
            if s == 0:
                out_ref[pl.ds(me * M_PER, M_PER),
                        pl.ds(t * TILE_N, TILE_N)] = res
            else:
                send_buf[s, :, t * TILE_N:(t + 1) * TILE_N] = (
                    res.astype(jnp.bfloat16))
                send_rdma(s, t).start()

        for s in range(1, N_DEV):
            for t in range(N_T):
                send_rdma(s, t).wait_recv()
            origin = (me - s) % N_DEV
            out_ref[pl.ds(origin * M_PER, M_PER), :] = (
                recv_buf[s].astype(jnp.float32))

        for s in range(1, N_DEV):
            for t in range(N_T):
                send_rdma(s, t).wait_send()

    return pl.pallas_call(
        body,
        out_shape=jax.ShapeDtypeStruct((N_DEV * M_PER, N_PER), jnp.float32),
        in_specs=[
            pl.BlockSpec(memory_space=pltpu.VMEM),
            pl.BlockSpec(memory_space=pltpu.ANY),
        ],
        out_specs=pl.BlockSpec(memory_space=pltpu.VMEM),
        scratch_shapes=[
            pltpu.VMEM((M_PER, K), jnp.bfloat16),
            pltpu.VMEM((2, K, TILE_N), jnp.float32),
            pltpu.VMEM((N_DEV, M_PER, N_PER), jnp.bfloat16),
            pltpu.VMEM((N_DEV, M_PER, N_PER), jnp.bfloat16),
            pltpu.SemaphoreType.DMA((2,)),
            pltpu.SemaphoreType.DMA((N_DEV, N_T)),
            pltpu.SemaphoreType.DMA((N_DEV, N_T)),
        ],
        compiler_params=pltpu.CompilerParams(collective_id=0),
    )(x, w_mat)


# baseline (device time: 107371 ns/iter reference)
import jax
import jax.numpy as jnp
from jax import lax
from jax.experimental import pallas as pl
from jax.experimental.pallas import tpu as pltpu

N_DEV = 8
M_PER = 512
K = 4096
N_PER = 1024
TILE_N = 256
N_T = N_PER // TILE_N


def kernel(x, w_mat):
    def body(x_ref, w_hbm, out_ref, x_bf, w_buf, send_buf, recv_buf,
             w_sems, send_sems, recv_sems):
        me = lax.axis_index("i")

        barrier = pltpu.get_barrier_semaphore()
        for s in range(1, N_DEV):
            pl.semaphore_signal(
                barrier, inc=1,
                device_id=((me + s) % N_DEV,),
                device_id_type=pl.DeviceIdType.MESH,
            )
        pl.semaphore_wait(barrier, N_DEV - 1)

        x_bf[...] = x_ref[...].astype(jnp.bfloat16)

        seq = list(range(1, N_DEV)) + [0]
        tiles = [(s, t) for s in seq for t in range(N_T)]

        def w_col(s, t):
            return ((me + s) % N_DEV) * N_PER + t * TILE_N

        def w_load(k, slot):
            s, t = tiles[k]
            return pltpu.make_async_copy(
                w_hbm.at[:, pl.ds(w_col(s, t), TILE_N)],
                w_buf.at[slot],
                w_sems.at[slot],
            )

        def send_rdma(s, t):
            return pltpu.make_async_remote_copy(
                src_ref=send_buf.at[s, :, pl.ds(t * TILE_N, TILE_N)],
                dst_ref=recv_buf.at[s, :, pl.ds(t * TILE_N, TILE_N)],
                send_sem=send_sems.at[s, t],
                recv_sem=recv_sems.at[s, t],
                device_id=((me + s) % N_DEV,),
                device_id_type=pl.DeviceIdType.MESH,
            )

        w_load(0, 0).start()
        for k, (s, t) in enumerate(tiles):
            slot = k % 2
            if k + 1 < len(tiles):
                w_load(k + 1, (k + 1) % 2).start()
            w_load(k, slot).wait()
            res = jnp.dot(
                x_bf[...],
                w_buf[slot].astype(jnp.bfloat16),
                preferred_element_type=jnp.float32,
            )
            if s == 0:
                out_ref[pl.ds(me * M_PER, M_PER),
                        pl.ds(t * TILE_N, TILE_N)] = res
            else:
                send_buf[s, :, t * TILE_N:(t + 1) * TILE_N] = (
                    res.astype(jnp.bfloat16))
                send_rdma(s, t).start()

        for s in range(1, N_DEV):
            for t in range(N_T):
                send_rdma(s, t).wait_recv()
            origin = (me - s) % N_DEV
            out_ref[pl.ds(origin * M_PER, M_PER), :] = (
                recv_buf[s].astype(jnp.float32))

        for s in range(1, N_DEV):
            for t in range(N_T):
                send_rdma(s, t).wait_send()

    return pl.pallas_call(
        body,
        out_shape=jax.ShapeDtypeStruct((N_DEV * M_PER, N_PER), jnp.float32),
        in_specs=[
            pl.BlockSpec(memory_space=pltpu.VMEM),
            pl.BlockSpec(memory_space=pl.ANY),
        ],
        out_specs=pl.BlockSpec(memory_space=pltpu.VMEM),
        scratch_shapes=[
            pltpu.VMEM((M_PER, K), jnp.bfloat16),
            pltpu.VMEM((2, K, TILE_N), jnp.float32),
            pltpu.VMEM((N_DEV, M_PER, N_PER), jnp.bfloat16),
            pltpu.VMEM((N_DEV, M_PER, N_PER), jnp.bfloat16),
            pltpu.SemaphoreType.DMA((2,)),
            pltpu.SemaphoreType.DMA((N_DEV, N_T)),
            pltpu.SemaphoreType.DMA((N_DEV, N_T)),
        ],
        compiler_params=pltpu.CompilerParams(
            collective_id=0,
            vmem_limit_bytes=60 * 1024 * 1024,
        ),
    )(x, w_mat)
